# baseline (speedup 1.0000x reference)
import jax
import jax.numpy as jnp
from jax.experimental import pallas as pl
from jax.experimental.pallas import tpu as pltpu

_NCLS = 1000
_NS = 5
_HID = 512
_XD = 64
_POS_INF = 1e6
_CBLK = 8
_DN = (((1,), (0,)), ((), ()))


def _fgc_kernel(x_ref, g_ref, w1_ref, b1_ref, w2_ref, b2_ref, out_ref, cent_ref):
    w1 = w1_ref[:]
    w2 = w2_ref[:]
    b1 = b1_ref[:]
    xfsum = jnp.zeros((_NCLS, _XD), jnp.float32)
    for s in range(_NS):
        g_s = g_ref[pl.ds(s * _NCLS, _NCLS), :]
        h_s = jnp.maximum(jax.lax.dot_general(g_s, w1, _DN) + b1, 0.0)
        xfsum = xfsum + jax.lax.dot_general(h_s, w2, _DN)
    cent_ref[:] = xfsum * jnp.float32(1.0 / _NS) + b2_ref[:]

    x = x_ref[:]

    def body(i, carry):
        bd, bi = carry
        c = cent_ref[pl.ds(i * _CBLK, _CBLK), :]
        diff = c[None, :, :] - x[:, None, :]
        d = jnp.sum(diff * diff, axis=-1)
        ld = jnp.min(d, axis=1, keepdims=True)
        lane = jax.lax.broadcasted_iota(jnp.int32, d.shape, 1)
        li = jnp.min(
            jnp.where(d == ld, lane + i * _CBLK, jnp.int32(2 ** 30)),
            axis=1, keepdims=True)
        upd = ld < bd
        return jnp.where(upd, ld, bd), jnp.where(upd, li, bi)

    bd0 = jnp.full((x.shape[0], 1), jnp.float32(jnp.inf))
    bi0 = jnp.zeros((x.shape[0], 1), jnp.int32)
    _, bi = jax.lax.fori_loop(0, _NCLS // _CBLK, body, (bd0, bi0))
    col = jax.lax.broadcasted_iota(jnp.int32, (x.shape[0], _NCLS), 1)
    out_ref[:] = jnp.where(col == bi, jnp.float32(_POS_INF), jnp.float32(0.0))


def kernel(x, attrs, z, G_W1, G_b1, G_W2, G_b2):
    g = jnp.concatenate([z, jnp.tile(attrs, (_NS, 1))], axis=1)
    return pl.pallas_call(
        _fgc_kernel,
        out_shape=jax.ShapeDtypeStruct((x.shape[0], _NCLS), jnp.float32),
        scratch_shapes=[pltpu.VMEM((_NCLS, _XD), jnp.float32)],
        compiler_params=pltpu.CompilerParams(vmem_limit_bytes=64 * 1024 * 1024),
    )(x, g, G_W1, G_b1.reshape(1, _HID), G_W2, G_b2.reshape(1, _XD))


# speedup vs baseline: 9.1067x; 9.1067x over previous
import jax
import jax.numpy as jnp
from jax.experimental import pallas as pl
from jax.experimental.pallas import tpu as pltpu

_NCLS = 1000
_NS = 5
_HID = 512
_XD = 64
_B = 1024
_POS_INF = 1e6
_DN = (((1,), (0,)), ((), ()))
_HI = jax.lax.Precision.HIGHEST


def _fgc_kernel(x_ref, xt_ref, g_ref, w1_ref, b1_ref, w2_ref, b2_ref, out_ref):
    w1 = w1_ref[:]
    w2 = w2_ref[:]
    b1 = b1_ref[:]
    xfsum = jnp.zeros((_NCLS, _XD), jnp.float32)
    for s in range(_NS):
        g_s = g_ref[pl.ds(s * _NCLS, _NCLS), :]
        h_s = jnp.maximum(jax.lax.dot_general(g_s, w1, _DN) + b1, 0.0)
        xfsum = xfsum + jax.lax.dot_general(h_s, w2, _DN)
    cent = xfsum * jnp.float32(1.0 / _NS) + b2_ref[:]

    cn = jnp.sum(cent * cent, axis=1, keepdims=True)
    cx = jax.lax.dot_general(cent, xt_ref[:], _DN, precision=_HI)
    s_hat = cn - (cx + cx)

    big = jnp.int32(2 ** 30)
    row = jax.lax.broadcasted_iota(jnp.int32, s_hat.shape, 0)
    v1 = jnp.min(s_hat, axis=0, keepdims=True)
    i1 = jnp.min(jnp.where(s_hat == v1, row, big), axis=0, keepdims=True)
    masked = jnp.where(row == i1, jnp.float32(jnp.inf), s_hat)
    v2 = jnp.min(masked, axis=0, keepdims=True)
    i2 = jnp.min(jnp.where(masked == v2, row, big), axis=0, keepdims=True)

    i1c = jnp.transpose(i1.astype(jnp.float32)).astype(jnp.int32)
    i2c = jnp.transpose(i2.astype(jnp.float32)).astype(jnp.int32)

    x = x_ref[:]
    col = jax.lax.broadcasted_iota(jnp.int32, (_B, _NCLS), 1)
    oh1 = (col == i1c).astype(jnp.float32)
    c1 = jax.lax.dot_general(oh1, cent, _DN, precision=_HI)
    oh2 = (col == i2c).astype(jnp.float32)
    c2 = jax.lax.dot_general(oh2, cent, _DN, precision=_HI)
    diff1 = c1 - x
    d1 = jnp.sum(diff1 * diff1, axis=-1, keepdims=True)
    diff2 = c2 - x
    d2 = jnp.sum(diff2 * diff2, axis=-1, keepdims=True)
    w = jnp.where(d1 < d2, i1c,
                  jnp.where(d2 < d1, i2c, jnp.minimum(i1c, i2c)))
    out_ref[:] = jnp.where(col == w, jnp.float32(_POS_INF), jnp.float32(0.0))


def kernel(x, attrs, z, G_W1, G_b1, G_W2, G_b2):
    g = jnp.concatenate([z, jnp.tile(attrs, (_NS, 1))], axis=1)
    return pl.pallas_call(
        _fgc_kernel,
        out_shape=jax.ShapeDtypeStruct((x.shape[0], _NCLS), jnp.float32),
        compiler_params=pltpu.CompilerParams(vmem_limit_bytes=64 * 1024 * 1024),
    )(x, x.T, g, G_W1, G_b1.reshape(1, _HID), G_W2, G_b2.reshape(1, _XD))


# speedup vs baseline: 11.4700x; 1.2595x over previous
import jax
import jax.numpy as jnp
from jax.experimental import pallas as pl
from jax.experimental.pallas import tpu as pltpu

_NCLS = 1000
_NS = 5
_HID = 512
_XD = 64
_B = 1024
_POS_INF = 1e6
_DN = (((1,), (0,)), ((), ()))
_HI = jax.lax.Precision.HIGHEST


def _fgc_kernel(x_ref, xt_ref, z_ref, attrs_ref, w1_ref, b1_ref, w2_ref, b2_ref,
                out_ref):
    w1 = w1_ref[:]
    w2 = w2_ref[:]
    b1 = b1_ref[:]
    attrs = attrs_ref[:]
    xfsum = jnp.zeros((_NCLS, _XD), jnp.float32)
    for s in range(_NS):
        z_s = z_ref[pl.ds(s * _NCLS, _NCLS), :]
        g_s = jnp.concatenate([z_s, attrs], axis=1)
        h_s = jnp.maximum(jax.lax.dot_general(g_s, w1, _DN) + b1, 0.0)
        xfsum = xfsum + jax.lax.dot_general(h_s, w2, _DN)
    cent = xfsum * jnp.float32(1.0 / _NS) + b2_ref[:]

    cn = jnp.sum(cent * cent, axis=1, keepdims=True)
    cx = jax.lax.dot_general(cent, xt_ref[:], _DN, precision=_HI)
    s_hat = cn - (cx + cx)

    big = jnp.int32(2 ** 30)
    row = jax.lax.broadcasted_iota(jnp.int32, s_hat.shape, 0)
    v1 = jnp.min(s_hat, axis=0, keepdims=True)
    i1 = jnp.min(jnp.where(s_hat == v1, row, big), axis=0, keepdims=True)
    masked = jnp.where(row == i1, jnp.float32(jnp.inf), s_hat)
    v2 = jnp.min(masked, axis=0, keepdims=True)
    i2 = jnp.min(jnp.where(masked == v2, row, big), axis=0, keepdims=True)

    i1c = jnp.transpose(i1.astype(jnp.float32)).astype(jnp.int32)
    i2c = jnp.transpose(i2.astype(jnp.float32)).astype(jnp.int32)

    x = x_ref[:]
    col = jax.lax.broadcasted_iota(jnp.int32, (_B, _NCLS), 1)
    oh1 = (col == i1c).astype(jnp.float32)
    c1 = jax.lax.dot_general(oh1, cent, _DN, precision=_HI)
    oh2 = (col == i2c).astype(jnp.float32)
    c2 = jax.lax.dot_general(oh2, cent, _DN, precision=_HI)
    diff1 = c1 - x
    d1 = jnp.sum(diff1 * diff1, axis=-1, keepdims=True)
    diff2 = c2 - x
    d2 = jnp.sum(diff2 * diff2, axis=-1, keepdims=True)
    w = jnp.where(d1 < d2, i1c,
                  jnp.where(d2 < d1, i2c, jnp.minimum(i1c, i2c)))
    out_ref[:] = jnp.where(col == w, jnp.float32(_POS_INF), jnp.float32(0.0))


def kernel(x, attrs, z, G_W1, G_b1, G_W2, G_b2):
    return pl.pallas_call(
        _fgc_kernel,
        out_shape=jax.ShapeDtypeStruct((x.shape[0], _NCLS), jnp.float32),
        compiler_params=pltpu.CompilerParams(vmem_limit_bytes=64 * 1024 * 1024),
    )(x, x.T, z, attrs, G_W1, G_b1.reshape(1, _HID), G_W2, G_b2.reshape(1, _XD))
